# baseline (speedup 1.0000x reference)
import jax
import jax.numpy as jnp
import numpy as np
from jax import lax
from jax.experimental import pallas as pl
from jax.experimental.pallas import tpu as pltpu
from jax.experimental.pallas import tpu_sc as plsc

_ROWS = 128
_N = 32768
_L = 16
_NV = _N // _L
_NQ = 8
_QV = _NV // _NQ
_NC, _NS = 2, 16
_NW = _NC * _NS
_RPW = _ROWS // _NW
_NB = 256
_SIGN = np.int32(-(2 ** 31))


def _fwd(ki):
    return jnp.where(ki < 0, ~ki, ki ^ _SIGN)


def _inv(u):
    return jnp.where(u < 0, u ^ _SIGN, ~u)


def _sort_body(x_hbm, out_hbm, bufa, bufb, *cnts):
    wid = lax.axis_index("s") * _NC + lax.axis_index("c")
    lanes = lax.iota(jnp.int32, _L)
    ones = jnp.ones((_L,), jnp.int32)
    zeros = jnp.zeros((_L,), jnp.int32)
    x_i = x_hbm.bitcast(jnp.int32)
    out_i = out_hbm.bitcast(jnp.int32)

    def do_row(j, _):
        row = wid * _RPW + j
        pltpu.sync_copy(x_i.at[row], bufa)

        for p in range(4):
            src = bufa if p % 2 == 0 else bufb
            dst = bufb if p % 2 == 0 else bufa
            sh = 8 * p
            first = p == 0
            last = p == 3

            def zero(i):
                for k in range(2):
                    for c in cnts:
                        c[pl.ds((2 * i + k) * _L, _L)] = zeros

            plsc.parallel_loop(0, _NB // 2, unroll=4)(zero)

            def count(v):
                us = [src[pl.ds((q * _QV + 2 * v + k) * _L, _L)]
                      for q in range(_NQ) for k in range(2)]
                if first:
                    us = [_fwd(u) for u in us]
                idxs = [((u >> sh) & 0xFF) * _L + lanes for u in us]
                for q, c in enumerate(cnts):
                    for k in range(2):
                        plsc.addupdate_scatter(c, [idxs[2 * q + k]], ones)

            plsc.parallel_loop(0, _QV // 2, unroll=4)(count)

            def scan(i, carry):
                sl = pl.ds(i * _L, _L)
                hs = [c[sl] for c in cnts]
                t01 = hs[0] + hs[1]
                t23 = hs[2] + hs[3]
                t45 = hs[4] + hs[5]
                t67 = hs[6] + hs[7]
                tot = (t01 + t23) + (t45 + t67)
                e = carry + plsc.cumsum(tot) - tot
                for q, c in enumerate(cnts):
                    c[sl] = e
                    e = e + hs[q]
                return carry + jnp.sum(tot)

            plsc.parallel_loop(
                0, _NB, unroll=4,
                carry=jnp.zeros((_L,), jnp.int32))(scan)

            def permute(v, _):
                us = [src[pl.ds((q * _QV + 2 * v + k) * _L, _L)]
                      for q in range(_NQ) for k in range(2)]
                if first:
                    us = [_fwd(u) for u in us]
                idxs = [((u >> sh) & 0xFF) * _L + lanes for u in us]
                s0 = [plsc.load_gather(cnts[q], [idxs[2 * q]])
                      for q in range(_NQ)]
                for q, c in enumerate(cnts):
                    plsc.addupdate_scatter(c, [idxs[2 * q]], ones)
                s1 = [plsc.load_gather(cnts[q], [idxs[2 * q + 1]])
                      for q in range(_NQ)]
                for q, c in enumerate(cnts):
                    plsc.addupdate_scatter(c, [idxs[2 * q + 1]], ones)
                ss = [s0[q] if k == 0 else s1[q]
                      for q in range(_NQ) for k in range(2)]
                if last:
                    for e in range(2 * _NQ):
                        plsc.store_scatter(dst, [ss[e]], _inv(us[e]))
                else:
                    for e in range(2 * _NQ):
                        phys = ((ss[e] & (_NV - 1)) << 4) | (ss[e] >> 11)
                        plsc.store_scatter(dst, [phys], us[e])
                return 0

            lax.fori_loop(0, _QV // 2, permute, 0)

        pltpu.sync_copy(bufa, out_i.at[row])
        return 0

    lax.fori_loop(0, _RPW, do_row, 0)


def kernel(x):
    mesh = plsc.VectorSubcoreMesh(core_axis_name="c", subcore_axis_name="s")
    f = pl.kernel(
        _sort_body,
        out_type=jax.ShapeDtypeStruct((_ROWS, _N), jnp.float32),
        mesh=mesh,
        compiler_params=pltpu.CompilerParams(needs_layout_passes=False),
        scratch_types=[
            pltpu.VMEM((_N,), jnp.int32),
            pltpu.VMEM((_N,), jnp.int32),
        ] + [
            pltpu.VMEM((_NB * _L,), jnp.int32)
            for _ in range(_NQ)
        ],
    )
    return f(x)


# speedup vs baseline: 1.0031x; 1.0031x over previous
import jax
import jax.numpy as jnp
import numpy as np
from jax import lax
from jax.experimental import pallas as pl
from jax.experimental.pallas import tpu as pltpu
from jax.experimental.pallas import tpu_sc as plsc

_ROWS = 128
_N = 32768
_L = 16
_NV = _N // _L
_NQ = 8
_QV = _NV // _NQ
_NC, _NS = 2, 16
_NW = _NC * _NS
_RPW = _ROWS // _NW
_NB = 256
_SIGN = np.int32(-(2 ** 31))


def _fwd(ki):
    return jnp.where(ki < 0, ~ki, ki ^ _SIGN)


def _inv(u):
    return jnp.where(u < 0, u ^ _SIGN, ~u)


def _sort_body(x_hbm, out_hbm, bufa, bufb, *cnts):
    wid = lax.axis_index("s") * _NC + lax.axis_index("c")
    lanes = lax.iota(jnp.int32, _L)
    ones = jnp.ones((_L,), jnp.int32)
    sixteens = jnp.full((_L,), 16, jnp.int32)
    zeros = jnp.zeros((_L,), jnp.int32)
    x_i = x_hbm.bitcast(jnp.int32)
    out_i = out_hbm.bitcast(jnp.int32)

    def do_row(j, _):
        row = wid * _RPW + j
        pltpu.sync_copy(x_i.at[row], bufa)

        for p in range(4):
            src = bufa if p % 2 == 0 else bufb
            dst = bufb if p % 2 == 0 else bufa
            sh = 8 * p
            first = p == 0
            last = p == 3

            def zero(i):
                for k in range(2):
                    for c in cnts:
                        c[pl.ds((2 * i + k) * _L, _L)] = zeros

            plsc.parallel_loop(0, _NB // 2, unroll=2)(zero)

            def count(v):
                us = [src[pl.ds((q * _QV + 2 * v + k) * _L, _L)]
                      for q in range(_NQ) for k in range(2)]
                if first:
                    us = [_fwd(u) for u in us]
                idxs = [((u >> sh) & 0xFF) * _L + lanes for u in us]
                for q, c in enumerate(cnts):
                    for k in range(2):
                        plsc.addupdate_scatter(c, [idxs[2 * q + k]], ones)

            plsc.parallel_loop(0, _QV // 2, unroll=2)(count)

            def scan(i, carry):
                sl = pl.ds(i * _L, _L)
                hs = [c[sl] for c in cnts]
                t01 = hs[0] + hs[1]
                t23 = hs[2] + hs[3]
                t45 = hs[4] + hs[5]
                t67 = hs[6] + hs[7]
                tot = (t01 + t23) + (t45 + t67)
                e = carry + plsc.cumsum(tot) - tot
                for q, c in enumerate(cnts):
                    c[sl] = e << 4
                    e = e + hs[q]
                return carry + jnp.sum(tot)

            plsc.parallel_loop(
                0, _NB, unroll=2,
                carry=jnp.zeros((_L,), jnp.int32))(scan)

            def permute(v, _):
                us = [src[pl.ds((q * _QV + 2 * v + k) * _L, _L)]
                      for q in range(_NQ) for k in range(2)]
                if first:
                    us = [_fwd(u) for u in us]
                idxs = [((u >> sh) & 0xFF) * _L + lanes for u in us]
                s0 = [plsc.load_gather(cnts[q], [idxs[2 * q]])
                      for q in range(_NQ)]
                for q, c in enumerate(cnts):
                    plsc.addupdate_scatter(c, [idxs[2 * q]], sixteens)
                s1 = [plsc.load_gather(cnts[q], [idxs[2 * q + 1]])
                      for q in range(_NQ)]
                for q, c in enumerate(cnts):
                    plsc.addupdate_scatter(c, [idxs[2 * q + 1]], sixteens)
                ss = [s0[q] if k == 0 else s1[q]
                      for q in range(_NQ) for k in range(2)]
                if last:
                    for e in range(2 * _NQ):
                        plsc.store_scatter(dst, [ss[e] >> 4], _inv(us[e]))
                else:
                    for e in range(2 * _NQ):
                        phys = (ss[e] & (_N - 16)) | (ss[e] >> 15)
                        plsc.store_scatter(dst, [phys], us[e])
                return 0

            lax.fori_loop(0, _QV // 2, permute, 0)

        pltpu.sync_copy(bufa, out_i.at[row])
        return 0

    lax.fori_loop(0, _RPW, do_row, 0)


def kernel(x):
    mesh = plsc.VectorSubcoreMesh(core_axis_name="c", subcore_axis_name="s")
    f = pl.kernel(
        _sort_body,
        out_type=jax.ShapeDtypeStruct((_ROWS, _N), jnp.float32),
        mesh=mesh,
        compiler_params=pltpu.CompilerParams(needs_layout_passes=False),
        scratch_types=[
            pltpu.VMEM((_N,), jnp.int32),
            pltpu.VMEM((_N,), jnp.int32),
        ] + [
            pltpu.VMEM((_NB * _L,), jnp.int32)
            for _ in range(_NQ)
        ],
    )
    return f(x)


# speedup vs baseline: 1.0297x; 1.0265x over previous
import jax
import jax.numpy as jnp
import numpy as np
from jax import lax
from jax.experimental import pallas as pl
from jax.experimental.pallas import tpu as pltpu
from jax.experimental.pallas import tpu_sc as plsc

_ROWS = 128
_N = 32768
_L = 16
_NV = _N // _L
_NQ = 8
_QV = _NV // _NQ
_NC, _NS = 2, 16
_NW = _NC * _NS
_RPW = _ROWS // _NW
_NB = 256
_SIGN = np.int32(-(2 ** 31))


def _fwd(ki):
    return jnp.where(ki < 0, ~ki, ki ^ _SIGN)


def _inv(u):
    return jnp.where(u < 0, u ^ _SIGN, ~u)


def _sort_body(x_hbm, out_hbm, bufa, bufb, *cnts):
    wid = lax.axis_index("s") * _NC + lax.axis_index("c")
    lanes = lax.iota(jnp.int32, _L)
    ones = jnp.ones((_L,), jnp.int32)
    zeros = jnp.zeros((_L,), jnp.int32)
    x_i = x_hbm.bitcast(jnp.int32)
    out_i = out_hbm.bitcast(jnp.int32)

    def do_row(j, _):
        row = wid * _RPW + j
        pltpu.sync_copy(x_i.at[row], bufa)

        for p in range(4):
            src = bufa if p % 2 == 0 else bufb
            dst = bufb if p % 2 == 0 else bufa
            sh = 8 * p
            first = p == 0
            last = p == 3

            def zero(i):
                for k in range(2):
                    for c in cnts:
                        c[pl.ds((2 * i + k) * _L, _L)] = zeros

            plsc.parallel_loop(0, _NB // 2, unroll=2)(zero)

            def count(v):
                us = [src[pl.ds((q * _QV + 2 * v + k) * _L, _L)]
                      for q in range(_NQ) for k in range(2)]
                if first:
                    us = [_fwd(u) for u in us]
                idxs = [((u >> sh) & 0xFF) * _L + lanes for u in us]
                for q, c in enumerate(cnts):
                    for k in range(2):
                        plsc.addupdate_scatter(c, [idxs[2 * q + k]], ones)

            plsc.parallel_loop(0, _QV // 2, unroll=2)(count)

            def scan(i, carry):
                sl = pl.ds(i * _L, _L)
                hs = [c[sl] for c in cnts]
                t01 = hs[0] + hs[1]
                t23 = hs[2] + hs[3]
                t45 = hs[4] + hs[5]
                t67 = hs[6] + hs[7]
                tot = (t01 + t23) + (t45 + t67)
                e = carry + plsc.cumsum(tot) - tot
                for q, c in enumerate(cnts):
                    c[sl] = e
                    e = e + hs[q]
                return carry + jnp.sum(tot)

            plsc.parallel_loop(
                0, _NB, unroll=2,
                carry=jnp.zeros((_L,), jnp.int32))(scan)

            def permute(v, _):
                us = [src[pl.ds((q * _QV + 2 * v + k) * _L, _L)]
                      for q in range(_NQ) for k in range(2)]
                if first:
                    us = [_fwd(u) for u in us]
                idxs = [((u >> sh) & 0xFF) * _L + lanes for u in us]
                s0 = [plsc.load_gather(cnts[q], [idxs[2 * q]])
                      for q in range(_NQ)]
                for q, c in enumerate(cnts):
                    plsc.addupdate_scatter(c, [idxs[2 * q]], ones)
                s1 = [plsc.load_gather(cnts[q], [idxs[2 * q + 1]])
                      for q in range(_NQ)]
                for q, c in enumerate(cnts):
                    plsc.addupdate_scatter(c, [idxs[2 * q + 1]], ones)
                ss = [s0[q] if k == 0 else s1[q]
                      for q in range(_NQ) for k in range(2)]
                if last:
                    for e in range(2 * _NQ):
                        plsc.store_scatter(dst, [ss[e]], _inv(us[e]))
                else:
                    for e in range(2 * _NQ):
                        phys = ((ss[e] & (_NV - 1)) << 4) | (ss[e] >> 11)
                        plsc.store_scatter(dst, [phys], us[e])
                return 0

            lax.fori_loop(0, _QV // 2, permute, 0)

        pltpu.sync_copy(bufa, out_i.at[row])
        return 0

    lax.fori_loop(0, _RPW, do_row, 0)


def kernel(x):
    mesh = plsc.VectorSubcoreMesh(core_axis_name="c", subcore_axis_name="s")
    f = pl.kernel(
        _sort_body,
        out_type=jax.ShapeDtypeStruct((_ROWS, _N), jnp.float32),
        mesh=mesh,
        compiler_params=pltpu.CompilerParams(needs_layout_passes=False),
        scratch_types=[
            pltpu.VMEM((_N,), jnp.int32),
            pltpu.VMEM((_N,), jnp.int32),
        ] + [
            pltpu.VMEM((_NB * _L,), jnp.int32)
            for _ in range(_NQ)
        ],
    )
    return f(x)
